# baseline (speedup 1.0000x reference)
import functools

import jax
import jax.numpy as jnp
from jax.experimental import pallas as pl
from jax.experimental.pallas import tpu as pltpu

_HI = jax.lax.Precision.HIGHEST
_BF = jnp.bfloat16
_F32 = jnp.float32

_B, _N, _D, _E, _CAP = 32, 8, 1024, 8, 2
_ED = _CAP * _D
_T = _B * _N
_SWB = 256
_S_SW = (_N * _D) // _SWB
_OB = 512
_NO = _ED // _OB
_S_FC = _E * _NO
_S_CH1 = _NO
_TOTAL = _S_SW + 2 * _S_FC + _S_CH1 + 1


def _bdot(a, b_t):
    return jnp.dot(a.astype(_BF), b_t.astype(_BF).T,
                   preferred_element_type=_F32)


def _gelu(v):
    return 0.5 * v * (1.0 + jax.lax.erf(v * 0.7071067811865475))


def _router(x2, emb):
    logits = jnp.dot(x2.astype(_BF), emb.astype(_BF).T,
                     preferred_element_type=_F32)
    col = jax.lax.broadcasted_iota(jnp.int32, (_T, _E), 1)
    m1 = jnp.max(logits, axis=1, keepdims=True)
    i1 = jnp.min(jnp.where(logits == m1, col, _E), axis=1, keepdims=True)
    masked = jnp.where(col == i1, -jnp.inf, logits)
    m2 = jnp.max(masked, axis=1, keepdims=True)
    i2 = jnp.min(jnp.where(masked == m2, col, _E), axis=1, keepdims=True)
    t = jax.lax.broadcasted_iota(jnp.int32, (_T, 1), 0)
    base = t - t % _N
    src = jnp.concatenate([(base + i1).astype(_F32),
                           (base + i2).astype(_F32)], axis=1)
    q = (t % _B) * _N + t // _B
    colT = jax.lax.broadcasted_iota(jnp.int32, (_T, _T), 1)
    perm = (colT == q).astype(_F32)
    srcp = jnp.dot(perm, src, preferred_element_type=_F32, precision=_HI)
    s1 = srcp[:, 0:1].astype(jnp.int32)
    s2 = srcp[:, 1:2].astype(jnp.int32)
    g1 = jnp.dot((colT == s1).astype(_F32), x2,
                 preferred_element_type=_F32, precision=_HI)
    g2 = jnp.dot((colT == s2).astype(_F32), x2,
                 preferred_element_type=_F32, precision=_HI)
    return g1, g2


def _mega_kernel(x_tok_ref, x_flat_ref, emb_ref,
                 sw1_ref, sw1b_ref, sw2_ref, sw2b_ref,
                 f1w_ref, f1b_ref, f2w_ref, f2b_ref,
                 c1w_ref, c1b_ref, c2w_ref, c2b_ref,
                 out_ref,
                 sel_ref, h1_ref, wts_ref, swacc_ref, ws_ref, hid_ref):
    s = pl.program_id(0)

    @pl.when(s == 0)
    def _router_step():
        g1, g2 = _router(x_tok_ref[:], emb_ref[:])
        sel_ref[:, 0] = g1[:, 0:_OB].reshape(_E, _B, _OB)
        sel_ref[:, 1] = g1[:, _OB:_D].reshape(_E, _B, _OB)
        sel_ref[:, 2] = g2[:, 0:_OB].reshape(_E, _B, _OB)
        sel_ref[:, 3] = g2[:, _OB:_D].reshape(_E, _B, _OB)

    @pl.when(s < _S_SW)
    def _sw_step():
        part = _bdot(x_flat_ref[:], sw1_ref[:])
        h = _gelu(part + sw1b_ref[:])
        contrib = _bdot(h, sw2_ref[:])

        @pl.when(s == 0)
        def _():
            swacc_ref[:] = contrib

        @pl.when(s > 0)
        def _():
            swacc_ref[:] = swacc_ref[:] + contrib

        @pl.when(s == _S_SW - 1)
        def _():
            lg = swacc_ref[:] + sw2b_ref[:]
            m = jnp.max(lg, axis=1, keepdims=True)
            ez = jnp.exp(lg - m)
            wts_ref[:] = ez / jnp.sum(ez, axis=1, keepdims=True)

    @pl.when(jnp.logical_and(s >= _S_SW, s < _S_SW + _S_FC))
    def _fc1_step():
        t = s - _S_SW
        e = t // _NO
        o = t % _NO
        w = f1w_ref[0]
        acc = _bdot(sel_ref[e, 0], w[:, 0 * _OB:1 * _OB])
        acc += _bdot(sel_ref[e, 1], w[:, 1 * _OB:2 * _OB])
        acc += _bdot(sel_ref[e, 2], w[:, 2 * _OB:3 * _OB])
        acc += _bdot(sel_ref[e, 3], w[:, 3 * _OB:4 * _OB])
        h1_ref[e, o] = _gelu(acc + f1b_ref[0])

    @pl.when(jnp.logical_and(s >= _S_SW + _S_FC, s < _S_SW + 2 * _S_FC))
    def _fc2_step():
        t = s - (_S_SW + _S_FC)
        e = t // _NO
        o = t % _NO
        w = f2w_ref[0]
        acc = _bdot(h1_ref[e, 0], w[:, 0 * _OB:1 * _OB])
        acc += _bdot(h1_ref[e, 1], w[:, 1 * _OB:2 * _OB])
        acc += _bdot(h1_ref[e, 2], w[:, 2 * _OB:3 * _OB])
        acc += _bdot(h1_ref[e, 3], w[:, 3 * _OB:4 * _OB])
        r = acc + f2b_ref[0]
        ecol = jax.lax.broadcasted_iota(jnp.int32, (_E, 1), 0)
        wcol = jnp.dot(wts_ref[:], (ecol == e).astype(_F32),
                       preferred_element_type=_F32, precision=_HI)
        contrib = r * wcol

        @pl.when(e == 0)
        def _():
            ws_ref[o] = contrib

        @pl.when(e > 0)
        def _():
            ws_ref[o] = ws_ref[o] + contrib

    @pl.when(jnp.logical_and(s >= _S_SW + 2 * _S_FC, s < _TOTAL - 1))
    def _ch1_step():
        t = s - (_S_SW + 2 * _S_FC)
        w = c1w_ref[:]
        acc = _bdot(ws_ref[0], w[:, 0 * _OB:1 * _OB])
        acc += _bdot(ws_ref[1], w[:, 1 * _OB:2 * _OB])
        acc += _bdot(ws_ref[2], w[:, 2 * _OB:3 * _OB])
        acc += _bdot(ws_ref[3], w[:, 3 * _OB:4 * _OB])
        hid_ref[t] = _gelu(acc + c1b_ref[:])

    @pl.when(s == _TOTAL - 1)
    def _ch2_step():
        w = c2w_ref[:]
        acc = _bdot(hid_ref[0], w[:, 0 * _OB:1 * _OB])
        acc += _bdot(hid_ref[1], w[:, 1 * _OB:2 * _OB])
        acc += _bdot(hid_ref[2], w[:, 2 * _OB:3 * _OB])
        acc += _bdot(hid_ref[3], w[:, 3 * _OB:4 * _OB])
        out_ref[:] = acc + c2b_ref[:]


def kernel(x, expert_emb, exp_fc1_w, exp_fc1_b, exp_fc2_w, exp_fc2_b,
           sw_fc1_w, sw_fc1_b, sw_fc2_w, sw_fc2_b,
           ch_fc1_w, ch_fc1_b, ch_fc2_w, ch_fc2_b):
    bsz, ntok, dim = x.shape
    assert (bsz, ntok, dim) == (_B, _N, _D)
    ncls = ch_fc2_w.shape[0]

    x_tok = x.reshape(_T, _D)
    x_flat = x.reshape(_B, _N * _D)

    S_SW, S_FC, NO = _S_SW, _S_FC, _NO

    def sw_idx(s):
        return jnp.minimum(s, S_SW - 1)

    def fc1_idx(s):
        t = jnp.clip(s - S_SW, 0, S_FC - 1)
        return t // NO, t % NO

    def fc2_idx(s):
        t = jnp.clip(s - S_SW - S_FC, 0, S_FC - 1)
        return t // NO, t % NO

    def ch1_idx(s):
        return jnp.clip(s - S_SW - 2 * S_FC, 0, _S_CH1 - 1)

    out = pl.pallas_call(
        _mega_kernel,
        grid=(_TOTAL,),
        in_specs=[
            pl.BlockSpec((_T, _D), lambda s: (0, 0)),
            pl.BlockSpec((_B, _N * _D), lambda s: (0, 0)),
            pl.BlockSpec((_E, _D), lambda s: (0, 0)),
            pl.BlockSpec((_SWB, _N * _D), lambda s: (sw_idx(s), 0)),
            pl.BlockSpec((1, _SWB), lambda s: (0, sw_idx(s))),
            pl.BlockSpec((_E, _SWB), lambda s: (0, sw_idx(s))),
            pl.BlockSpec((1, _E), lambda s: (0, 0)),
            pl.BlockSpec((1, _OB, _ED),
                         lambda s: (*fc1_idx(s), 0)),
            pl.BlockSpec((1, 1, _OB),
                         lambda s: (fc1_idx(s)[0], 0, fc1_idx(s)[1])),
            pl.BlockSpec((1, _OB, _ED),
                         lambda s: (*fc2_idx(s), 0)),
            pl.BlockSpec((1, 1, _OB),
                         lambda s: (fc2_idx(s)[0], 0, fc2_idx(s)[1])),
            pl.BlockSpec((_OB, _ED), lambda s: (ch1_idx(s), 0)),
            pl.BlockSpec((1, _OB), lambda s: (0, ch1_idx(s))),
            pl.BlockSpec((ncls, _ED), lambda s: (0, 0)),
            pl.BlockSpec((1, ncls), lambda s: (0, 0)),
        ],
        out_specs=pl.BlockSpec((_B, ncls), lambda s: (0, 0)),
        out_shape=jax.ShapeDtypeStruct((_B, ncls), _F32),
        scratch_shapes=[
            pltpu.VMEM((_E, _NO, _B, _OB), _F32),
            pltpu.VMEM((_E, _NO, _B, _OB), _F32),
            pltpu.VMEM((_B, _E), _F32),
            pltpu.VMEM((_B, _E), _F32),
            pltpu.VMEM((_NO, _B, _OB), _F32),
            pltpu.VMEM((_NO, _B, _OB), _F32),
        ],
        compiler_params=pltpu.CompilerParams(
            vmem_limit_bytes=58 * 1024 * 1024),
    )(x_tok, x_flat, expert_emb,
      sw_fc1_w, sw_fc1_b.reshape(1, -1), sw_fc2_w, sw_fc2_b.reshape(1, -1),
      exp_fc1_w, exp_fc1_b.reshape(_E, 1, _ED),
      exp_fc2_w, exp_fc2_b.reshape(_E, 1, _ED),
      ch_fc1_w, ch_fc1_b.reshape(1, -1), ch_fc2_w, ch_fc2_b.reshape(1, -1))
    return out


# speedup vs baseline: 1.0520x; 1.0520x over previous
import jax
import jax.numpy as jnp
from jax.experimental import pallas as pl
from jax.experimental.pallas import tpu as pltpu

_HI = jax.lax.Precision.HIGHEST


def _gelu(v):
    return 0.5 * v * (1.0 + jax.lax.erf(v * 0.7071067811865475))


def _router_kernel(x_ref, emb_ref, sel_ref, *, bsz, ntok, dim, nexp):
    T = bsz * ntok
    x2 = x_ref[:]
    logits = jnp.dot(x2.astype(jnp.bfloat16), emb_ref[:].astype(jnp.bfloat16).T,
                     preferred_element_type=jnp.float32)
    col = jax.lax.broadcasted_iota(jnp.int32, (T, nexp), 1)
    m1 = jnp.max(logits, axis=1, keepdims=True)
    i1 = jnp.min(jnp.where(logits == m1, col, nexp), axis=1, keepdims=True)
    masked = jnp.where(col == i1, -jnp.inf, logits)
    m2 = jnp.max(masked, axis=1, keepdims=True)
    i2 = jnp.min(jnp.where(masked == m2, col, nexp), axis=1, keepdims=True)
    t = jax.lax.broadcasted_iota(jnp.int32, (T, 1), 0)
    base = t - t % ntok
    src = jnp.concatenate([(base + i1).astype(jnp.float32),
                           (base + i2).astype(jnp.float32)], axis=1)
    q = (t % bsz) * ntok + t // bsz
    colT = jax.lax.broadcasted_iota(jnp.int32, (T, T), 1)
    perm = (colT == q).astype(jnp.float32)
    srcp = jnp.dot(perm, src, preferred_element_type=jnp.float32,
                   precision=_HI)
    s1 = srcp[:, 0:1].astype(jnp.int32)
    s2 = srcp[:, 1:2].astype(jnp.int32)
    oh1 = (colT == s1).astype(jnp.float32)
    oh2 = (colT == s2).astype(jnp.float32)
    g1 = jnp.dot(oh1, x2, preferred_element_type=jnp.float32, precision=_HI)
    g2 = jnp.dot(oh2, x2, preferred_element_type=jnp.float32, precision=_HI)
    sel = jnp.concatenate([g1, g2], axis=1)
    sel_ref[:] = sel.reshape(nexp, bsz, 2 * dim)


def _sw_kernel(x_ref, w1t_ref, w1b_ref, b1t_ref, b1b_ref, w2t_ref, w2b_ref,
               b2_ref, wts_ref, acc_ref):
    s = pl.program_id(0)
    xb = x_ref[:].astype(jnp.bfloat16)
    ht = _gelu(jnp.dot(xb, w1t_ref[:].astype(jnp.bfloat16).T,
                       preferred_element_type=jnp.float32) + b1t_ref[:])
    hb = _gelu(jnp.dot(xb, w1b_ref[:].astype(jnp.bfloat16).T,
                       preferred_element_type=jnp.float32) + b1b_ref[:])
    contrib = (jnp.dot(ht.astype(jnp.bfloat16), w2t_ref[:].astype(jnp.bfloat16).T,
                       preferred_element_type=jnp.float32)
               + jnp.dot(hb.astype(jnp.bfloat16), w2b_ref[:].astype(jnp.bfloat16).T,
                         preferred_element_type=jnp.float32))

    @pl.when(s == 0)
    def _():
        acc_ref[:] = contrib

    @pl.when(s > 0)
    def _():
        acc_ref[:] = acc_ref[:] + contrib

    @pl.when(s == pl.num_programs(0) - 1)
    def _():
        logits = acc_ref[:] + b2_ref[:]
        m = jnp.max(logits, axis=1, keepdims=True)
        ez = jnp.exp(logits - m)
        wts_ref[:] = ez / jnp.sum(ez, axis=1, keepdims=True)


def _fc1_kernel(sel_ref, w_ref, b_ref, h_ref):
    sb = sel_ref[0].astype(jnp.bfloat16)
    wb = w_ref[0].astype(jnp.bfloat16)
    h = jnp.dot(sb, wb.T, preferred_element_type=jnp.float32) + b_ref[0]
    h_ref[0] = _gelu(h)


def _fc2_kernel(h_ref, w_ref, b_ref, wts_ref, out_ref, *, nexp):
    e = pl.program_id(0)
    hb = h_ref[0].astype(jnp.bfloat16)
    wb = w_ref[0].astype(jnp.bfloat16)
    r = jnp.dot(hb, wb.T, preferred_element_type=jnp.float32) + b_ref[0]
    ecol = jax.lax.broadcasted_iota(jnp.int32, (nexp, 1), 0)
    onehot = (ecol == e).astype(jnp.float32)
    wcol = jnp.dot(wts_ref[:], onehot, preferred_element_type=jnp.float32,
                   precision=_HI)
    contrib = r * wcol

    @pl.when(e == 0)
    def _():
        out_ref[:] = contrib

    @pl.when(e > 0)
    def _():
        out_ref[:] = out_ref[:] + contrib


def _head_kernel(ws_ref, w1_ref, b1_ref, w2_ref, b2_ref, out_ref):
    wsb = ws_ref[:].astype(jnp.bfloat16)
    h = jnp.dot(wsb, w1_ref[:].astype(jnp.bfloat16).T,
                preferred_element_type=jnp.float32) + b1_ref[:]
    hb = _gelu(h).astype(jnp.bfloat16)
    out_ref[:] = jnp.dot(hb, w2_ref[:].astype(jnp.bfloat16).T,
                         preferred_element_type=jnp.float32) + b2_ref[:]


def kernel(x, expert_emb, exp_fc1_w, exp_fc1_b, exp_fc2_w, exp_fc2_b,
           sw_fc1_w, sw_fc1_b, sw_fc2_w, sw_fc2_b,
           ch_fc1_w, ch_fc1_b, ch_fc2_w, ch_fc2_b):
    import functools
    bsz, ntok, dim = x.shape
    nexp = expert_emb.shape[0]
    ed = exp_fc1_w.shape[1]
    ncls = ch_fc2_w.shape[0]
    f32 = jnp.float32

    x_tok = x.reshape(bsz * ntok, dim)
    x_flat = x.reshape(bsz, ntok * dim)

    sel = pl.pallas_call(
        functools.partial(_router_kernel, bsz=bsz, ntok=ntok, dim=dim,
                          nexp=nexp),
        out_shape=jax.ShapeDtypeStruct((nexp, bsz, ed), f32),
    )(x_tok, expert_emb)

    SWB = 256
    nsteps = (ntok * dim) // (2 * SWB)
    b1_2d = sw_fc1_b.reshape(1, -1)
    wts = pl.pallas_call(
        _sw_kernel,
        grid=(nsteps,),
        in_specs=[
            pl.BlockSpec((bsz, ntok * dim), lambda s: (0, 0)),
            pl.BlockSpec((SWB, ntok * dim), lambda s: (s, 0)),
            pl.BlockSpec((SWB, ntok * dim), lambda s: (s + 16, 0)),
            pl.BlockSpec((1, SWB), lambda s: (0, s)),
            pl.BlockSpec((1, SWB), lambda s: (0, s + 16)),
            pl.BlockSpec((nexp, SWB), lambda s: (0, s)),
            pl.BlockSpec((nexp, SWB), lambda s: (0, s + 16)),
            pl.BlockSpec((1, nexp), lambda s: (0, 0)),
        ],
        out_specs=pl.BlockSpec((bsz, nexp), lambda s: (0, 0)),
        out_shape=jax.ShapeDtypeStruct((bsz, nexp), f32),
        scratch_shapes=[pltpu.VMEM((bsz, nexp), f32)],
        compiler_params=pltpu.CompilerParams(
            vmem_limit_bytes=60 * 1024 * 1024),
    )(x_flat, sw_fc1_w, sw_fc1_w, b1_2d, b1_2d, sw_fc2_w, sw_fc2_w,
      sw_fc2_b.reshape(1, -1))

    h1 = pl.pallas_call(
        _fc1_kernel,
        grid=(nexp,),
        in_specs=[
            pl.BlockSpec((1, bsz, ed), lambda e: (e, 0, 0)),
            pl.BlockSpec((1, ed, ed), lambda e: (e, 0, 0)),
            pl.BlockSpec((1, 1, ed), lambda e: (e, 0, 0)),
        ],
        out_specs=pl.BlockSpec((1, bsz, ed), lambda e: (e, 0, 0)),
        out_shape=jax.ShapeDtypeStruct((nexp, bsz, ed), f32),
        compiler_params=pltpu.CompilerParams(
            vmem_limit_bytes=60 * 1024 * 1024),
    )(sel, exp_fc1_w, exp_fc1_b.reshape(nexp, 1, ed))

    ws = pl.pallas_call(
        functools.partial(_fc2_kernel, nexp=nexp),
        grid=(nexp,),
        in_specs=[
            pl.BlockSpec((1, bsz, ed), lambda e: (e, 0, 0)),
            pl.BlockSpec((1, ed, ed), lambda e: (e, 0, 0)),
            pl.BlockSpec((1, 1, ed), lambda e: (e, 0, 0)),
            pl.BlockSpec((bsz, nexp), lambda e: (0, 0)),
        ],
        out_specs=pl.BlockSpec((bsz, ed), lambda e: (0, 0)),
        out_shape=jax.ShapeDtypeStruct((bsz, ed), f32),
        compiler_params=pltpu.CompilerParams(
            vmem_limit_bytes=60 * 1024 * 1024),
    )(h1, exp_fc2_w, exp_fc2_b.reshape(nexp, 1, ed), wts)

    out = pl.pallas_call(
        _head_kernel,
        in_specs=[
            pl.BlockSpec((bsz, ed), lambda: (0, 0)),
            pl.BlockSpec((ed, ed), lambda: (0, 0)),
            pl.BlockSpec((1, ed), lambda: (0, 0)),
            pl.BlockSpec((ncls, ed), lambda: (0, 0)),
            pl.BlockSpec((1, ncls), lambda: (0, 0)),
        ],
        out_specs=pl.BlockSpec((bsz, ncls), lambda: (0, 0)),
        out_shape=jax.ShapeDtypeStruct((bsz, ncls), f32),
        compiler_params=pltpu.CompilerParams(
            vmem_limit_bytes=60 * 1024 * 1024),
    )(ws, ch_fc1_w, ch_fc1_b.reshape(1, -1), ch_fc2_w,
      ch_fc2_b.reshape(1, -1))
    return out
